# baseline (speedup 1.0000x reference)
import functools

import jax
import jax.numpy as jnp
from jax import lax
from jax.experimental import pallas as pl
from jax.experimental.pallas import tpu as pltpu
from jax.experimental.pallas import tpu_sc as plsc

N = 10000
D = 128
HD = D // 2
NC = 2
NS = 16
NW = NC * NS
L = 16
CH = 128
NP = 10240
RPT = NP // NS
NBUF = 4
EPS = 1e-5

_mesh = plsc.VectorSubcoreMesh(core_axis_name="c", subcore_axis_name="s")


@functools.cache
def _deg_kernel(tch):
    cpt = tch // NW

    def body(dst, out, dst_v, ones_v, buf_v, deg):
        c = lax.axis_index("c")
        s = lax.axis_index("s")
        wid = c * NS + s
        pltpu.sync_copy(dst.at[pl.ds(wid * cpt, cpt)], dst_v)
        for j in range(CH // L):
            ones_v[pl.ds(j * L, L)] = jnp.ones((L,), jnp.float32)

        def zbuf(i, carry):
            buf_v[pl.ds(i * L, L)] = jnp.zeros((L,), jnp.float32)
            return carry

        lax.fori_loop(0, RPT // L, zbuf, 0)
        pltpu.sync_copy(buf_v, deg.at[pl.ds(s * RPT, RPT)])
        plsc.subcore_barrier()

        def step(k, carry):
            pltpu.sync_copy(ones_v, deg.at[dst_v.at[k]], add=True)
            return carry

        lax.fori_loop(0, cpt, step, 0)
        plsc.subcore_barrier()
        pltpu.sync_copy(deg.at[pl.ds(s * RPT, RPT)], buf_v)
        pltpu.sync_copy(buf_v, out.at[c, pl.ds(s * RPT, RPT)])

    return pl.kernel(
        body,
        out_type=jax.ShapeDtypeStruct((NC, NP), jnp.float32),
        mesh=_mesh,
        scratch_types=[
            pltpu.VMEM((cpt, CH), jnp.int32),
            pltpu.VMEM((CH,), jnp.float32),
            pltpu.VMEM((RPT,), jnp.float32),
            pltpu.VMEM_SHARED((NP,), jnp.float32),
        ],
    )


@functools.cache
def _prop_kernel(tch):
    cpt = tch // NS

    def body(hs, srcs, dst, out, src_v, dst_v, rows, gsems, ssems, acc):
        c = lax.axis_index("c")
        s = lax.axis_index("s")
        pltpu.sync_copy(srcs.at[c, pl.ds(s * cpt, cpt)], src_v)
        pltpu.sync_copy(dst.at[pl.ds(s * cpt, cpt)], dst_v)

        zv = jnp.zeros((L,), jnp.float32)

        def zrow(i, carry):
            for j in range(HD // L):
                rows[0][i, pl.ds(j * L, L)] = zv
            return carry

        lax.fori_loop(0, CH, zrow, 0)
        for r in range(RPT // CH):
            pltpu.sync_copy(rows[0], acc.at[pl.ds(s * RPT + r * CH, CH)])
        plsc.subcore_barrier()

        def gather(k, b):
            pltpu.async_copy(hs.at[src_v.at[k]], rows[b], gsems[b])

        def gwait(k, b):
            pltpu.make_async_copy(hs.at[src_v.at[k]], rows[b], gsems[b]).wait()

        def scat(k, b):
            pltpu.async_copy(rows[b], acc.at[dst_v.at[k]], ssems[b], add=True)

        def swait(k, b):
            pltpu.make_async_copy(rows[b], acc.at[dst_v.at[k]], ssems[b]).wait()

        for b in range(NBUF):
            gather(b, b)

        def step(i, carry):
            for b in range(NBUF):
                k = i * NBUF + b
                gwait(k, b)
                scat(k, b)

            @pl.when(i + 1 < cpt // NBUF)
            def _():
                for b in range(NBUF):
                    k = i * NBUF + b
                    swait(k, b)
                    gather(k + NBUF, b)

            return carry

        lax.fori_loop(0, cpt // NBUF, step, 0)
        for b in range(NBUF):
            swait(cpt - NBUF + b, b)
        plsc.subcore_barrier()

        base = s * RPT
        for r in range(RPT // CH):
            pltpu.sync_copy(acc.at[pl.ds(base + r * CH, CH)], rows[0])
            pltpu.sync_copy(rows[0], out.at[c, pl.ds(base + r * CH, CH)])

    return pl.kernel(
        body,
        out_type=jax.ShapeDtypeStruct((NC, NP, HD), jnp.float32),
        mesh=_mesh,
        scratch_types=[
            pltpu.VMEM((cpt, CH), jnp.int32),
            pltpu.VMEM((cpt, CH), jnp.int32),
            [pltpu.VMEM((CH, HD), jnp.float32) for _ in range(NBUF)],
            [pltpu.SemaphoreType.DMA for _ in range(NBUF)],
            [pltpu.SemaphoreType.DMA for _ in range(NBUF)],
            pltpu.VMEM_SHARED((NP, HD), jnp.float32),
        ],
        compiler_params=pltpu.CompilerParams(use_tc_tiling_on_sc=False),
    )


def _halved(h):
    return jnp.concatenate([h[:, :HD], h[:, HD:]], axis=0)


def _tc_head(deg2, x, W1, dis_o, dinv_o, h_o, hs_o):
    deg = deg2[:, 0:1] + deg2[:, 1:2] + 1.0
    dis = lax.rsqrt(deg)
    dinv = 1.0 / deg
    dis_o[...] = dis
    dinv_o[...] = dinv
    h = jnp.dot(x[...], W1[...], preferred_element_type=jnp.float32)
    h_o[...] = h
    hsc = h * dis
    hs_o[0:N, :] = hsc[:, 0:HD]
    hs_o[N:2 * N, :] = hsc[:, HD:D]


def _assemble(acc, h, dis, dinv, b):
    p = jnp.concatenate([acc[0, :N, :], acc[1, :N, :]], axis=1)
    return p * dis[...] + h[...] * dinv[...] + b[...]


def _tc_mid(acc, h, dis, dinv, b, g, be, W, hn_o, hns_o):
    p = _assemble(acc, h, dis, dinv, b)
    mean = jnp.mean(p, axis=0, keepdims=True)
    cent = p - mean
    var = jnp.mean(cent * cent, axis=0, keepdims=True)
    y = jnp.maximum(cent * lax.rsqrt(var + EPS) * g[...] + be[...], 0.0)
    hn = jnp.dot(y, W[...], preferred_element_type=jnp.float32)
    hn_o[...] = hn
    hsc = hn * dis[...]
    hns_o[0:N, :] = hsc[:, 0:HD]
    hns_o[N:2 * N, :] = hsc[:, HD:D]


def _tc_tail(acc, h, dis, dinv, b, out_o):
    out_o[...] = _assemble(acc, h, dis, dinv, b)


_head_call = pl.pallas_call(
    _tc_head,
    out_shape=(
        jax.ShapeDtypeStruct((N, 1), jnp.float32),
        jax.ShapeDtypeStruct((N, 1), jnp.float32),
        jax.ShapeDtypeStruct((N, D), jnp.float32),
        jax.ShapeDtypeStruct((2 * N, HD), jnp.float32),
    ),
)

_mid_call = pl.pallas_call(
    _tc_mid,
    out_shape=(
        jax.ShapeDtypeStruct((N, D), jnp.float32),
        jax.ShapeDtypeStruct((2 * N, HD), jnp.float32),
    ),
)

_tail_call = pl.pallas_call(
    _tc_tail,
    out_shape=jax.ShapeDtypeStruct((N, D), jnp.float32),
)


def kernel(x, edge_index, W1, b1, g1, be1, W2, b2, g2, be2, W3, b3):
    E = edge_index.shape[1]
    tch = -(-E // CH)
    tch += (-tch) % (NS * 8)
    EP = tch * CH
    pad = EP - E
    src = jnp.concatenate(
        [edge_index[0], jnp.zeros((pad,), edge_index.dtype)]).reshape(-1, CH)
    dst = jnp.concatenate(
        [edge_index[1], jnp.full((pad,), N, edge_index.dtype)]).reshape(-1, CH)
    srcs = jnp.stack([src, src + N])

    deg2 = _deg_kernel(tch)(dst)
    deg_n = deg2[:, :N].T

    dis, dinv, h1, h1s = _head_call(deg_n, x, W1)
    acc = _prop_kernel(tch)(h1s, srcs, dst)
    h2, h2s = _mid_call(acc, h1, dis, dinv,
                        b1.reshape(1, D), g1.reshape(1, D), be1.reshape(1, D),
                        W2)
    acc = _prop_kernel(tch)(h2s, srcs, dst)
    h3, h3s = _mid_call(acc, h2, dis, dinv,
                        b2.reshape(1, D), g2.reshape(1, D), be2.reshape(1, D),
                        W3)
    acc = _prop_kernel(tch)(h3s, srcs, dst)
    out = _tail_call(acc, h3, dis, dinv, b3.reshape(1, D))
    return out


# speedup vs baseline: 1.0638x; 1.0638x over previous
import functools

import jax
import jax.numpy as jnp
from jax import lax
from jax.experimental import pallas as pl
from jax.experimental.pallas import tpu as pltpu
from jax.experimental.pallas import tpu_sc as plsc

N = 10000
D = 128
HD = D // 2
NC = 2
NS = 16
NW = NC * NS
L = 16
CH = 128
NP = 10240
RPT = NP // NS
NBUF = 2
RPC = 2
EPS = 1e-5

_mesh = plsc.VectorSubcoreMesh(core_axis_name="c", subcore_axis_name="s")


@functools.cache
def _deg_kernel(tch):
    cpt = tch // NW

    def body(dst, out, dst_v, ones_v, buf_v, deg):
        c = lax.axis_index("c")
        s = lax.axis_index("s")
        wid = c * NS + s
        pltpu.sync_copy(dst.at[pl.ds(wid * cpt, cpt)], dst_v)
        for j in range(CH // L):
            ones_v[pl.ds(j * L, L)] = jnp.ones((L,), jnp.float32)

        def zbuf(i, carry):
            buf_v[pl.ds(i * L, L)] = jnp.zeros((L,), jnp.float32)
            return carry

        lax.fori_loop(0, RPT // L, zbuf, 0)
        pltpu.sync_copy(buf_v, deg.at[pl.ds(s * RPT, RPT)])
        plsc.subcore_barrier()

        def step(k, carry):
            pltpu.sync_copy(ones_v, deg.at[dst_v.at[k]], add=True)
            return carry

        lax.fori_loop(0, cpt, step, 0)
        plsc.subcore_barrier()
        pltpu.sync_copy(deg.at[pl.ds(s * RPT, RPT)], buf_v)
        pltpu.sync_copy(buf_v, out.at[c, pl.ds(s * RPT, RPT)])

    return pl.kernel(
        body,
        out_type=jax.ShapeDtypeStruct((NC, NP), jnp.float32),
        mesh=_mesh,
        scratch_types=[
            pltpu.VMEM((cpt, CH), jnp.int32),
            pltpu.VMEM((CH,), jnp.float32),
            pltpu.VMEM((RPT,), jnp.float32),
            pltpu.VMEM_SHARED((NP,), jnp.float32),
        ],
    )


@functools.cache
def _prop_kernel(tch):
    cpt = tch // NS

    spt = cpt // RPC
    SC_ = RPC * CH

    def body(hs, srcs, dst, out, src_v, dst_v, rows, gsems, ssems, acc):
        c = lax.axis_index("c")
        s = lax.axis_index("s")
        pltpu.sync_copy(srcs.at[c, pl.ds(s * spt, spt)], src_v)
        pltpu.sync_copy(dst.at[pl.ds(s * spt, spt)], dst_v)

        zv = jnp.zeros((L,), jnp.float32)

        def zrow(i, carry):
            for j in range(HD // L):
                rows[0][i, pl.ds(j * L, L)] = zv
            return carry

        lax.fori_loop(0, CH, zrow, 0)
        for r in range(RPT // CH):
            pltpu.sync_copy(rows[0].at[pl.ds(0, CH)],
                            acc.at[pl.ds(s * RPT + r * CH, CH)])
        plsc.subcore_barrier()

        def gather(k, b):
            pltpu.async_copy(hs.at[src_v.at[k]], rows[b], gsems[b])

        def gwait(k, b):
            pltpu.make_async_copy(hs.at[src_v.at[k]], rows[b], gsems[b]).wait()

        def scat(k, b):
            pltpu.async_copy(rows[b], acc.at[dst_v.at[k]], ssems[b], add=True)

        def swait(k, b):
            pltpu.make_async_copy(rows[b], acc.at[dst_v.at[k]],
                                  ssems[b]).wait()

        for b in range(NBUF):
            gather(b, b)

        def step(i, carry):
            for b in range(NBUF):
                k = i * NBUF + b
                gwait(k, b)
                scat(k, b)

                @pl.when(i + 1 < spt // NBUF)
                def _():
                    swait(k, b)
                    gather(k + NBUF, b)

            return carry

        lax.fori_loop(0, spt // NBUF, step, 0)
        for b in range(NBUF):
            swait(spt - NBUF + b, b)
        plsc.subcore_barrier()

        base = s * RPT
        for r in range(RPT // CH):
            pltpu.sync_copy(acc.at[pl.ds(base + r * CH, CH)],
                            rows[0].at[pl.ds(0, CH)])
            pltpu.sync_copy(rows[0].at[pl.ds(0, CH)],
                            out.at[c, pl.ds(base + r * CH, CH)])

    return pl.kernel(
        body,
        out_type=jax.ShapeDtypeStruct((NC, NP, HD), jnp.float32),
        mesh=_mesh,
        scratch_types=[
            pltpu.VMEM((spt, RPC * CH), jnp.int32),
            pltpu.VMEM((spt, RPC * CH), jnp.int32),
            [pltpu.VMEM((RPC * CH, HD), jnp.float32) for _ in range(NBUF)],
            [pltpu.SemaphoreType.DMA for _ in range(NBUF)],
            [pltpu.SemaphoreType.DMA for _ in range(NBUF)],
            pltpu.VMEM_SHARED((NP, HD), jnp.float32),
        ],
        compiler_params=pltpu.CompilerParams(use_tc_tiling_on_sc=False),
    )


def _halved(h):
    return jnp.concatenate([h[:, :HD], h[:, HD:]], axis=0)


def _tc_head(deg2, x, W1, dis_o, dinv_o, h_o, hs_o):
    deg = deg2[:, 0:1] + deg2[:, 1:2] + 1.0
    dis = lax.rsqrt(deg)
    dinv = 1.0 / deg
    dis_o[...] = dis
    dinv_o[...] = dinv
    h = jnp.dot(x[...], W1[...], preferred_element_type=jnp.float32)
    h_o[...] = h
    hsc = h * dis
    hs_o[0:N, :] = hsc[:, 0:HD]
    hs_o[N:2 * N, :] = hsc[:, HD:D]


def _assemble(acc, h, dis, dinv, b):
    p = jnp.concatenate([acc[0, :N, :], acc[1, :N, :]], axis=1)
    return p * dis[...] + h[...] * dinv[...] + b[...]


def _tc_mid(acc, h, dis, dinv, b, g, be, W, hn_o, hns_o):
    p = _assemble(acc, h, dis, dinv, b)
    mean = jnp.mean(p, axis=0, keepdims=True)
    cent = p - mean
    var = jnp.mean(cent * cent, axis=0, keepdims=True)
    y = jnp.maximum(cent * lax.rsqrt(var + EPS) * g[...] + be[...], 0.0)
    hn = jnp.dot(y, W[...], preferred_element_type=jnp.float32)
    hn_o[...] = hn
    hsc = hn * dis[...]
    hns_o[0:N, :] = hsc[:, 0:HD]
    hns_o[N:2 * N, :] = hsc[:, HD:D]


def _tc_tail(acc, h, dis, dinv, b, out_o):
    out_o[...] = _assemble(acc, h, dis, dinv, b)


_head_call = pl.pallas_call(
    _tc_head,
    out_shape=(
        jax.ShapeDtypeStruct((N, 1), jnp.float32),
        jax.ShapeDtypeStruct((N, 1), jnp.float32),
        jax.ShapeDtypeStruct((N, D), jnp.float32),
        jax.ShapeDtypeStruct((2 * N, HD), jnp.float32),
    ),
)

_mid_call = pl.pallas_call(
    _tc_mid,
    out_shape=(
        jax.ShapeDtypeStruct((N, D), jnp.float32),
        jax.ShapeDtypeStruct((2 * N, HD), jnp.float32),
    ),
)

_tail_call = pl.pallas_call(
    _tc_tail,
    out_shape=jax.ShapeDtypeStruct((N, D), jnp.float32),
)


def kernel(x, edge_index, W1, b1, g1, be1, W2, b2, g2, be2, W3, b3):
    E = edge_index.shape[1]
    tch = -(-E // CH)
    tch += (-tch) % (NS * 8 * RPC)
    EP = tch * CH
    pad = EP - E
    src = jnp.concatenate(
        [edge_index[0], jnp.zeros((pad,), edge_index.dtype)])
    dst = jnp.concatenate(
        [edge_index[1], jnp.full((pad,), N, edge_index.dtype)])
    src_p = src.reshape(-1, RPC * CH)
    dst_p = dst.reshape(-1, RPC * CH)
    srcs = jnp.stack([src_p, src_p + N])

    deg2 = _deg_kernel(tch)(dst.reshape(-1, CH))
    deg_n = deg2[:, :N].T

    dis, dinv, h1, h1s = _head_call(deg_n, x, W1)
    acc = _prop_kernel(tch)(h1s, srcs, dst_p)
    h2, h2s = _mid_call(acc, h1, dis, dinv,
                        b1.reshape(1, D), g1.reshape(1, D), be1.reshape(1, D),
                        W2)
    acc = _prop_kernel(tch)(h2s, srcs, dst_p)
    h3, h3s = _mid_call(acc, h2, dis, dinv,
                        b2.reshape(1, D), g2.reshape(1, D), be2.reshape(1, D),
                        W3)
    acc = _prop_kernel(tch)(h3s, srcs, dst_p)
    out = _tail_call(acc, h3, dis, dinv, b3.reshape(1, D))
    return out


# speedup vs baseline: 1.7531x; 1.6480x over previous
import functools

import jax
import jax.numpy as jnp
from jax import lax
from jax.experimental import pallas as pl
from jax.experimental.pallas import tpu as pltpu
from jax.experimental.pallas import tpu_sc as plsc

N = 10000
D = 128
HD = D // 2
NC = 2
NS = 16
NW = NC * NS
L = 16
CH = 128
NP = 10240
RPT = NP // NS
NBUF = 2
RPC = 2
EPS = 1e-5

_mesh = plsc.VectorSubcoreMesh(core_axis_name="c", subcore_axis_name="s")


@functools.cache
def _deg_kernel(tch):
    cpt = tch // NW

    def body(dst, out, dst_v, ones_v, buf_v, deg):
        c = lax.axis_index("c")
        s = lax.axis_index("s")
        wid = c * NS + s
        pltpu.sync_copy(dst.at[pl.ds(wid * cpt, cpt)], dst_v)
        for j in range(CH // L):
            ones_v[pl.ds(j * L, L)] = jnp.ones((L,), jnp.float32)

        def zbuf(i, carry):
            buf_v[pl.ds(i * L, L)] = jnp.zeros((L,), jnp.float32)
            return carry

        lax.fori_loop(0, RPT // L, zbuf, 0)
        pltpu.sync_copy(buf_v, deg.at[pl.ds(s * RPT, RPT)])
        plsc.subcore_barrier()

        def step(k, carry):
            pltpu.sync_copy(ones_v, deg.at[dst_v.at[k]], add=True)
            return carry

        lax.fori_loop(0, cpt, step, 0)
        plsc.subcore_barrier()
        pltpu.sync_copy(deg.at[pl.ds(s * RPT, RPT)], buf_v)
        pltpu.sync_copy(buf_v, out.at[c, pl.ds(s * RPT, RPT)])

    return pl.kernel(
        body,
        out_type=jax.ShapeDtypeStruct((NC, NP), jnp.float32),
        mesh=_mesh,
        scratch_types=[
            pltpu.VMEM((cpt, CH), jnp.int32),
            pltpu.VMEM((CH,), jnp.float32),
            pltpu.VMEM((RPT,), jnp.float32),
            pltpu.VMEM_SHARED((NP,), jnp.float32),
        ],
    )


@functools.cache
def _prop_kernel(tch):
    cpt = tch // NS
    spt = cpt // RPC
    SC_ = RPC * CH

    def body(hs, srcp, dstp, out, sidx, didx, rows, isems, gsems, ssems,
             table, acc):
        c = lax.axis_index("c")
        s = lax.axis_index("s")

        for r in range(RPT // CH):
            off = s * RPT + r * CH
            pltpu.sync_copy(hs.at[pl.ds(c * NP + off, CH)],
                            rows[0].at[pl.ds(0, CH)])
            pltpu.sync_copy(rows[0].at[pl.ds(0, CH)], table.at[pl.ds(off, CH)])

        zv = jnp.zeros((L,), jnp.float32)

        def zrow(i, carry):
            for j in range(HD // L):
                rows[0][i, pl.ds(j * L, L)] = zv
            return carry

        lax.fori_loop(0, CH, zrow, 0)
        for r in range(RPT // CH):
            pltpu.sync_copy(rows[0].at[pl.ds(0, CH)],
                            acc.at[pl.ds(s * RPT + r * CH, CH)])
        plsc.subcore_barrier()

        def iload(k, m):
            pltpu.async_copy(srcp.at[s * spt + k], sidx[m], isems[m])
            pltpu.async_copy(dstp.at[s * spt + k], didx[m], isems[m])

        def iwait(k, m):
            pltpu.make_async_copy(srcp.at[s * spt + k], sidx[m],
                                  isems[m]).wait()
            pltpu.make_async_copy(dstp.at[s * spt + k], didx[m],
                                  isems[m]).wait()

        def gather(b, m):
            pltpu.async_copy(table.at[sidx[m]], rows[b], gsems[b])

        def gwait(b, m):
            pltpu.make_async_copy(table.at[sidx[m]], rows[b],
                                  gsems[b]).wait()

        def scat(b, m):
            pltpu.async_copy(rows[b], acc.at[didx[m]], ssems[b], add=True)

        def swait(b, m):
            pltpu.make_async_copy(rows[b], acc.at[didx[m]], ssems[b]).wait()

        NSLOT = 2 * NBUF
        for m in range(NBUF):
            iload(m, m)
        for m in range(NBUF):
            iwait(m, m)
            gather(m, m)

        def step(j, carry):
            for m in range(NSLOT):
                k = j * NSLOT + m
                b = m % NBUF
                mn = (m + NBUF) % NSLOT
                gwait(b, m)

                @pl.when(k + NBUF < spt)
                def _():
                    iload(k + NBUF, mn)

                scat(b, m)

                @pl.when(k + NBUF < spt)
                def _():
                    swait(b, m)
                    iwait(k + NBUF, mn)
                    gather(b, mn)

            return carry

        lax.fori_loop(0, spt // NSLOT, step, 0)
        for b in range(NBUF):
            swait(b, NBUF + b)
        plsc.subcore_barrier()

        base = s * RPT
        for r in range(RPT // CH):
            pltpu.sync_copy(acc.at[pl.ds(base + r * CH, CH)],
                            rows[0].at[pl.ds(0, CH)])
            pltpu.sync_copy(rows[0].at[pl.ds(0, CH)],
                            out.at[c, pl.ds(base + r * CH, CH)])

    return pl.kernel(
        body,
        out_type=jax.ShapeDtypeStruct((NC, NP, HD), jnp.float32),
        mesh=_mesh,
        scratch_types=[
            [pltpu.VMEM((SC_,), jnp.int32) for _ in range(2 * NBUF)],
            [pltpu.VMEM((SC_,), jnp.int32) for _ in range(2 * NBUF)],
            [pltpu.VMEM((SC_, HD), jnp.float32) for _ in range(NBUF)],
            [pltpu.SemaphoreType.DMA for _ in range(2 * NBUF)],
            [pltpu.SemaphoreType.DMA for _ in range(NBUF)],
            [pltpu.SemaphoreType.DMA for _ in range(NBUF)],
            pltpu.VMEM_SHARED((NP, HD), jnp.float32),
            pltpu.VMEM_SHARED((NP, HD), jnp.float32),
        ],
        compiler_params=pltpu.CompilerParams(use_tc_tiling_on_sc=False),
    )


def _halved(h):
    return jnp.concatenate([h[:, :HD], h[:, HD:]], axis=0)


def _tc_head(deg2, x, W1, dis_o, dinv_o, h_o, hs_o):
    deg = deg2[:, 0:1] + deg2[:, 1:2] + 1.0
    dis = lax.rsqrt(deg)
    dinv = 1.0 / deg
    dis_o[...] = dis
    dinv_o[...] = dinv
    h = jnp.dot(x[...], W1[...], preferred_element_type=jnp.float32)
    h_o[...] = h
    hsc = h * dis
    hs_o[0:N, :] = hsc[:, 0:HD]
    hs_o[NP:NP + N, :] = hsc[:, HD:D]


def _assemble(acc, h, dis, dinv, b):
    p = jnp.concatenate([acc[0, :N, :], acc[1, :N, :]], axis=1)
    return p * dis[...] + h[...] * dinv[...] + b[...]


def _tc_mid(acc, h, dis, dinv, b, g, be, W, hn_o, hns_o):
    p = _assemble(acc, h, dis, dinv, b)
    mean = jnp.mean(p, axis=0, keepdims=True)
    cent = p - mean
    var = jnp.mean(cent * cent, axis=0, keepdims=True)
    y = jnp.maximum(cent * lax.rsqrt(var + EPS) * g[...] + be[...], 0.0)
    hn = jnp.dot(y, W[...], preferred_element_type=jnp.float32)
    hn_o[...] = hn
    hsc = hn * dis[...]
    hns_o[0:N, :] = hsc[:, 0:HD]
    hns_o[NP:NP + N, :] = hsc[:, HD:D]


def _tc_tail(acc, h, dis, dinv, b, out_o):
    out_o[...] = _assemble(acc, h, dis, dinv, b)


_head_call = pl.pallas_call(
    _tc_head,
    out_shape=(
        jax.ShapeDtypeStruct((N, 1), jnp.float32),
        jax.ShapeDtypeStruct((N, 1), jnp.float32),
        jax.ShapeDtypeStruct((N, D), jnp.float32),
        jax.ShapeDtypeStruct((2 * NP, HD), jnp.float32),
    ),
)

_mid_call = pl.pallas_call(
    _tc_mid,
    out_shape=(
        jax.ShapeDtypeStruct((N, D), jnp.float32),
        jax.ShapeDtypeStruct((2 * NP, HD), jnp.float32),
    ),
)

_tail_call = pl.pallas_call(
    _tc_tail,
    out_shape=jax.ShapeDtypeStruct((N, D), jnp.float32),
)


def kernel(x, edge_index, W1, b1, g1, be1, W2, b2, g2, be2, W3, b3):
    E = edge_index.shape[1]
    tch = -(-E // CH)
    tch += (-tch) % (NS * 8 * RPC)
    EP = tch * CH
    pad = EP - E
    src = jnp.concatenate(
        [edge_index[0], jnp.zeros((pad,), edge_index.dtype)])
    dst = jnp.concatenate(
        [edge_index[1], jnp.full((pad,), N, edge_index.dtype)])
    src_p = src.reshape(-1, RPC * CH)
    dst_p = dst.reshape(-1, RPC * CH)

    deg2 = _deg_kernel(tch)(dst.reshape(-1, CH))
    deg_n = deg2[:, :N].T

    dis, dinv, h1, h1s = _head_call(deg_n, x, W1)
    acc = _prop_kernel(tch)(h1s, src_p, dst_p)
    h2, h2s = _mid_call(acc, h1, dis, dinv,
                        b1.reshape(1, D), g1.reshape(1, D), be1.reshape(1, D),
                        W2)
    acc = _prop_kernel(tch)(h2s, src_p, dst_p)
    h3, h3s = _mid_call(acc, h2, dis, dinv,
                        b2.reshape(1, D), g2.reshape(1, D), be2.reshape(1, D),
                        W3)
    acc = _prop_kernel(tch)(h3s, src_p, dst_p)
    out = _tail_call(acc, h3, dis, dinv, b3.reshape(1, D))
    return out
